# baseline (speedup 1.0000x reference)
import functools

import jax
import jax.numpy as jnp
from jax import lax
from jax.experimental import pallas as pl
from jax.experimental.pallas import tpu as pltpu
import jax.experimental.pallas.tpu_sc as plsc

NUM_ROWS = 100000
DIM = 4096
TOPK = 32

BLOCK_ROWS = 1024
NUM_BLOCKS = 98
PAD_ROWS = NUM_BLOCKS * BLOCK_ROWS
SUBS = PAD_ROWS // 128

NEG_INF = float("-inf")
BIG = 2 ** 30

NTILES = 16
CHUNK = PAD_ROWS // NTILES
CHUNK_PAD = 6400
NSLICE = CHUNK_PAD // 16
CAND = 48
MERGE = NTILES * CAND
MSLICE = MERGE // 16

LAST_CHUNK = NUM_ROWS - (NTILES - 1) * CHUNK


def _matvec_kernel(x_ref, w_ref, out_ref):
    i = pl.program_id(0)
    y = lax.dot_general(
        w_ref[...].astype(jnp.bfloat16), x_ref[...].astype(jnp.bfloat16),
        dimension_numbers=(((1,), (0,)), ((), ())),
        preferred_element_type=jnp.float32,
    )
    y2 = y.reshape(8, 128)
    sub = lax.broadcasted_iota(jnp.int32, (8, 128), 0)
    lane = lax.broadcasted_iota(jnp.int32, (8, 128), 1)
    g = i * BLOCK_ROWS + sub * 128 + lane
    out_ref[...] = jnp.where(g < NUM_ROWS, y2, NEG_INF)


def _matvec(x, W):
    return pl.pallas_call(
        _matvec_kernel,
        grid=(NUM_BLOCKS,),
        in_specs=[
            pl.BlockSpec((DIM,), lambda i: (0,)),
            pl.BlockSpec((BLOCK_ROWS, DIM), lambda i: (i, 0)),
        ],
        out_specs=pl.BlockSpec((8, 128), lambda i: (i, 0)),
        out_shape=jax.ShapeDtypeStruct((SUBS, 128), jnp.float32),
    )(x, W)


def _lane_iota():
    return lax.iota(jnp.int32, 16)


def _splat_i32(v):
    return jnp.broadcast_to(v.astype(jnp.int32), (16,))


def _splat_f32(v):
    return jnp.broadcast_to(v.astype(jnp.float32), (16,))


def _build_slice_max(buf, smax, n_groups):
    lanes = _lane_iota()

    def body(j, _):
        base_ids = (j * 16 + lanes) * 16
        m = plsc.load_gather(buf, [base_ids])
        for k in range(1, 16):
            m = jnp.maximum(m, plsc.load_gather(buf, [base_ids + k]))
        smax[pl.ds(j * 16, 16)] = m
        return 0

    lax.fori_loop(0, n_groups, body, 0, unroll=False)


def _select_topk(buf, smax, n_groups, n_iter, out_vals, out_inds, ind_of):
    lanes = _lane_iota()
    lane0 = lanes == 0

    def round_body(t, _):
        def scan(j, carry):
            m16, id16 = carry
            v = smax[pl.ds(j * 16, 16)]
            better = v > m16
            return (jnp.where(better, v, m16),
                    jnp.where(better, j * 16 + lanes, id16))

        m16, id16 = lax.fori_loop(
            0, n_groups, scan,
            (jnp.full((16,), NEG_INF, jnp.float32),
             jnp.full((16,), BIG, jnp.int32)),
            unroll=False)
        m = jnp.max(m16)
        s_star = jnp.min(jnp.where(m16 == m, id16, BIG))

        v = buf[pl.ds(s_star * 16, 16)]
        l_star = jnp.min(jnp.where(v == m, lanes, BIG))
        flat = s_star * 16 + l_star

        plsc.store_scatter(out_vals, [_splat_i32(t)], _splat_f32(m),
                           mask=lane0)
        plsc.store_scatter(out_inds, [_splat_i32(t)], ind_of(flat),
                           mask=lane0)

        plsc.store_scatter(buf, [_splat_i32(flat)],
                           jnp.full((16,), NEG_INF, jnp.float32), mask=lane0)
        v2 = buf[pl.ds(s_star * 16, 16)]
        plsc.store_scatter(smax, [_splat_i32(s_star)],
                           _splat_f32(jnp.max(v2)), mask=lane0)
        return 0

    lax.fori_loop(0, n_iter, round_body, 0, unroll=False)


def _sc_body(res_hbm, out_hbm, buf, smax, lvals, linds, mvals, minds, msmax,
             shv, shi, shsv, shsi):
    c = lax.axis_index("c")
    w = lax.axis_index("s")
    lanes = _lane_iota()
    base = w * CHUNK

    @pl.when(c == 0)
    def _local():
        pltpu.sync_copy(res_hbm.at[pl.ds(base, CHUNK)], buf.at[pl.ds(0, CHUNK)])
        for j in range((CHUNK_PAD - CHUNK) // 16):
            buf[pl.ds(CHUNK + j * 16, 16)] = jnp.full((16,), NEG_INF,
                                                      jnp.float32)
        for j in range(CAND // 16):
            lvals[pl.ds(j * 16, 16)] = jnp.full((16,), NEG_INF, jnp.float32)
            linds[pl.ds(j * 16, 16)] = jnp.full((16,), BIG, jnp.int32)

        _build_slice_max(buf, smax, NSLICE // 16)
        _select_topk(buf, smax, NSLICE // 16, TOPK + 1, lvals, linds,
                     lambda flat: _splat_i32(base + flat))

        pltpu.sync_copy(lvals, shv.at[pl.ds(w * CAND, CAND)])
        pltpu.sync_copy(linds, shi.at[pl.ds(w * CAND, CAND)])

    plsc.subcore_barrier()

    @pl.when((c == 0) & (w == 0))
    def _merge():
        pltpu.sync_copy(shv, mvals)
        pltpu.sync_copy(shi, minds)
        for j in range(CAND // 16):
            lvals[pl.ds(j * 16, 16)] = jnp.full((16,), NEG_INF, jnp.float32)
            linds[pl.ds(j * 16, 16)] = jnp.full((16,), BIG, jnp.int32)

        _build_slice_max(mvals, msmax, MSLICE // 16)
        _select_topk(mvals, msmax, MSLICE // 16, TOPK + 1, lvals, linds,
                     lambda flat: plsc.load_gather(minds, [_splat_i32(flat)]))

        v0 = lvals[pl.ds(0, 16)][0]
        vlast = lvals[pl.ds(TOPK, 16)][0]
        vlast16 = _splat_f32(vlast)
        inv16 = jnp.full((16,), 1.0, jnp.float32) / _splat_f32(v0 - vlast)
        for j in range(CAND // 16):
            sv = (lvals[pl.ds(j * 16, 16)] - vlast16) * inv16
            lvals[pl.ds(j * 16, 16)] = sv

        pltpu.sync_copy(lvals, shsv)
        pltpu.sync_copy(linds, shsi)

    plsc.subcore_barrier()

    @pl.when(c == 0)
    def _scatter_out():
        pltpu.sync_copy(shsv, lvals)
        pltpu.sync_copy(shsi, linds)

        def zero(j, _):
            buf[pl.ds(j * 16, 16)] = jnp.zeros((16,), jnp.float32)
            return 0

        lax.fori_loop(0, CHUNK_PAD // 16, zero, 0, unroll=False)

        for b in range(2):
            idx = linds[pl.ds(b * 16, 16)]
            val = lvals[pl.ds(b * 16, 16)]
            local = idx - base
            ok = (local >= 0) & (local < CHUNK)
            safe = jnp.minimum(jnp.maximum(local, 0), CHUNK - 1)
            plsc.store_scatter(buf, [safe], val, mask=ok)

        @pl.when(w < NTILES - 1)
        def _():
            pltpu.sync_copy(buf.at[pl.ds(0, CHUNK)],
                            out_hbm.at[pl.ds(base, CHUNK)])

        @pl.when(w == NTILES - 1)
        def _():
            pltpu.sync_copy(buf.at[pl.ds(0, LAST_CHUNK)],
                            out_hbm.at[pl.ds(base, LAST_CHUNK)])


_sc_topk_scatter = functools.partial(
    pl.kernel,
    out_type=jax.ShapeDtypeStruct((NUM_ROWS,), jnp.float32),
    mesh=plsc.VectorSubcoreMesh(core_axis_name="c", subcore_axis_name="s",
                                num_cores=2, num_subcores=NTILES),
    compiler_params=pltpu.CompilerParams(needs_layout_passes=False),
    scratch_types=[
        pltpu.VMEM((CHUNK_PAD,), jnp.float32),
        pltpu.VMEM((NSLICE,), jnp.float32),
        pltpu.VMEM((CAND,), jnp.float32),
        pltpu.VMEM((CAND,), jnp.int32),
        pltpu.VMEM((MERGE,), jnp.float32),
        pltpu.VMEM((MERGE,), jnp.int32),
        pltpu.VMEM((MSLICE,), jnp.float32),
        pltpu.VMEM_SHARED((MERGE,), jnp.float32),
        pltpu.VMEM_SHARED((MERGE,), jnp.int32),
        pltpu.VMEM_SHARED((CAND,), jnp.float32),
        pltpu.VMEM_SHARED((CAND,), jnp.int32),
    ],
)(_sc_body)


@jax.jit
def kernel(x, W):
    res = _matvec(x, W)
    return _sc_topk_scatter(res.reshape(PAD_ROWS))


# speedup vs baseline: 1.0077x; 1.0077x over previous
import functools

import jax
import jax.numpy as jnp
from jax import lax
from jax.experimental import pallas as pl
from jax.experimental.pallas import tpu as pltpu
import jax.experimental.pallas.tpu_sc as plsc

NUM_ROWS = 100000
DIM = 4096
TOPK = 32

BLOCK_ROWS = 1024
NUM_BLOCKS = 98
PAD_ROWS = NUM_BLOCKS * BLOCK_ROWS
SUBS = PAD_ROWS // 128

NEG_INF = float("-inf")
BIG = 2 ** 30

NTILES = 16
CHUNK = PAD_ROWS // NTILES
CHUNK_PAD = 6400
NSLICE = CHUNK_PAD // 16
SMAX_PAD = 512
NGROUP = SMAX_PAD // 16
CAND = 48
MERGE = NTILES * CAND
MSLICE = MERGE // 16
MSMAX_PAD = 256

LAST_CHUNK = NUM_ROWS - (NTILES - 1) * CHUNK


def _matvec_kernel(x_ref, w_ref, out_ref):
    i = pl.program_id(0)
    y = lax.dot_general(
        w_ref[...].astype(jnp.bfloat16), x_ref[...].astype(jnp.bfloat16),
        dimension_numbers=(((1,), (0,)), ((), ())),
        preferred_element_type=jnp.float32,
    )
    y2 = y.reshape(BLOCK_ROWS // 128, 128)
    sub = lax.broadcasted_iota(jnp.int32, (BLOCK_ROWS // 128, 128), 0)
    lane = lax.broadcasted_iota(jnp.int32, (BLOCK_ROWS // 128, 128), 1)
    g = i * BLOCK_ROWS + sub * 128 + lane
    out_ref[...] = jnp.where(g < NUM_ROWS, y2, NEG_INF)


def _matvec(x, W):
    return pl.pallas_call(
        _matvec_kernel,
        grid=(NUM_BLOCKS,),
        in_specs=[
            pl.BlockSpec((DIM,), lambda i: (0,)),
            pl.BlockSpec((BLOCK_ROWS, DIM), lambda i: (i, 0)),
        ],
        out_specs=pl.BlockSpec((BLOCK_ROWS // 128, 128), lambda i: (i, 0)),
        out_shape=jax.ShapeDtypeStruct((SUBS, 128), jnp.float32),
    )(x, W)


def _lanes():
    return lax.iota(jnp.int32, 16)


def _splat_i32(v):
    return jnp.broadcast_to(v.astype(jnp.int32), (16,))


def _splat_f32(v):
    return jnp.broadcast_to(v.astype(jnp.float32), (16,))


def _neg16():
    return jnp.full((16,), NEG_INF, jnp.float32)


def _fill(ref, start, n_vregs, value):
    for j in range(n_vregs):
        ref[pl.ds(start + j * 16, 16)] = value


def _build_slice_max(src, dst, n_groups, unroll=1):
    lanes = _lanes()

    def body(j, _):
        base_ids = (j * 16 + lanes) * 16
        m = plsc.load_gather(src, [base_ids])
        for k in range(1, 16):
            m = jnp.maximum(m, plsc.load_gather(src, [base_ids + k]))
        dst[pl.ds(j * 16, 16)] = m
        return 0

    lax.fori_loop(0, n_groups, body, 0, unroll=unroll)


def _locate(vec, m, ids):
    return jnp.min(jnp.where(vec == m, ids, BIG))


def _select_topk(buf, smax, l2, l2_vregs, n_iter, out_vals, out_inds, ind_of):
    lanes = _lanes()
    lane0 = lanes == 0

    def round_body(t, _):
        vs = [l2[pl.ds(j * 16, 16)] for j in range(l2_vregs)]
        m16 = vs[0]
        for v in vs[1:]:
            m16 = jnp.maximum(m16, v)
        m = jnp.max(m16)
        cand = jnp.where(vs[0] == m, lanes, BIG)
        for j in range(1, l2_vregs):
            cand = jnp.minimum(cand, jnp.where(vs[j] == m, j * 16 + lanes,
                                               BIG))
        g_star = jnp.min(cand)

        vg = smax[pl.ds(g_star * 16, 16)]
        s_star = g_star * 16 + _locate(vg, m, lanes)

        vb = buf[pl.ds(s_star * 16, 16)]
        flat = s_star * 16 + _locate(vb, m, lanes)

        plsc.store_scatter(out_vals, [_splat_i32(t)], _splat_f32(m),
                           mask=lane0)
        plsc.store_scatter(out_inds, [_splat_i32(t)], ind_of(flat),
                           mask=lane0)

        plsc.store_scatter(buf, [_splat_i32(flat)], _neg16(), mask=lane0)
        vb2 = buf[pl.ds(s_star * 16, 16)]
        plsc.store_scatter(smax, [_splat_i32(s_star)],
                           _splat_f32(jnp.max(vb2)), mask=lane0)
        vg2 = smax[pl.ds(g_star * 16, 16)]
        plsc.store_scatter(l2, [_splat_i32(g_star)],
                           _splat_f32(jnp.max(vg2)), mask=lane0)
        return 0

    lax.fori_loop(0, n_iter, round_body, 0, unroll=False)


def _sc_body(res_hbm, out_hbm, buf, zbuf, smax, l2, lvals, linds,
             mvals, minds, msmax, ml2, shv, shi, sem):
    c = lax.axis_index("c")
    w = lax.axis_index("s")
    base = w * CHUNK

    @pl.when(c == 0)
    def _local():
        cp = pltpu.async_copy(res_hbm.at[pl.ds(base, CHUNK)],
                              buf.at[pl.ds(0, CHUNK)], sem)
        zero16 = jnp.zeros((16,), jnp.float32)

        def zero_body(j, _):
            zbuf[pl.ds(j * 16, 16)] = zero16
            return 0

        lax.fori_loop(0, CHUNK_PAD // 16, zero_body, 0, unroll=8)
        _fill(buf, CHUNK, (CHUNK_PAD - CHUNK) // 16, _neg16())
        _fill(smax, 0, NGROUP, _neg16())
        _fill(l2, 0, NGROUP // 16, _neg16())
        _fill(lvals, 0, CAND // 16, _neg16())
        _fill(linds, 0, CAND // 16, jnp.full((16,), BIG, jnp.int32))
        cp.wait()

        _build_slice_max(buf, smax, NSLICE // 16, unroll=4)
        _build_slice_max(smax, l2, NGROUP // 16)
        _select_topk(buf, smax, l2, NGROUP // 16, TOPK + 1, lvals, linds,
                     lambda flat: _splat_i32(base + flat))

        pltpu.sync_copy(lvals, shv.at[pl.ds(w * CAND, CAND)])
        pltpu.sync_copy(linds, shi.at[pl.ds(w * CAND, CAND)])

    plsc.subcore_barrier()

    @pl.when(c == 0)
    def _merge_and_scatter():
        pltpu.sync_copy(shv, mvals)
        pltpu.sync_copy(shi, minds)
        _fill(msmax, 0, MSMAX_PAD // 16, _neg16())
        _fill(lvals, 0, CAND // 16, _neg16())
        _fill(linds, 0, CAND // 16, jnp.full((16,), BIG, jnp.int32))

        _build_slice_max(mvals, msmax, MSLICE // 16)
        _build_slice_max(msmax, ml2, 1)
        _select_topk(mvals, msmax, ml2, 1, TOPK + 1,
                     lvals, linds,
                     lambda flat: plsc.load_gather(minds, [_splat_i32(flat)]))

        v0 = lvals[pl.ds(0, 16)][0]
        vlast = lvals[pl.ds(TOPK, 16)][0]
        vlast16 = _splat_f32(vlast)
        inv16 = jnp.full((16,), 1.0, jnp.float32) / _splat_f32(v0 - vlast)

        for b in range(2):
            idx = linds[pl.ds(b * 16, 16)]
            val = (lvals[pl.ds(b * 16, 16)] - vlast16) * inv16
            local = idx - base
            ok = (local >= 0) & (local < CHUNK)
            safe = jnp.minimum(jnp.maximum(local, 0), CHUNK - 1)
            plsc.store_scatter(zbuf, [safe], val, mask=ok)

        @pl.when(w < NTILES - 1)
        def _():
            pltpu.sync_copy(zbuf.at[pl.ds(0, CHUNK)],
                            out_hbm.at[pl.ds(base, CHUNK)])

        @pl.when(w == NTILES - 1)
        def _():
            pltpu.sync_copy(zbuf.at[pl.ds(0, LAST_CHUNK)],
                            out_hbm.at[pl.ds(base, LAST_CHUNK)])


_sc_topk_scatter = functools.partial(
    pl.kernel,
    out_type=jax.ShapeDtypeStruct((NUM_ROWS,), jnp.float32),
    mesh=plsc.VectorSubcoreMesh(core_axis_name="c", subcore_axis_name="s",
                                num_cores=2, num_subcores=NTILES),
    compiler_params=pltpu.CompilerParams(needs_layout_passes=False),
    scratch_types=[
        pltpu.VMEM((CHUNK_PAD,), jnp.float32),
        pltpu.VMEM((CHUNK_PAD,), jnp.float32),
        pltpu.VMEM((SMAX_PAD,), jnp.float32),
        pltpu.VMEM((NGROUP,), jnp.float32),
        pltpu.VMEM((CAND,), jnp.float32),
        pltpu.VMEM((CAND,), jnp.int32),
        pltpu.VMEM((MERGE,), jnp.float32),
        pltpu.VMEM((MERGE,), jnp.int32),
        pltpu.VMEM((MSMAX_PAD,), jnp.float32),
        pltpu.VMEM((16,), jnp.float32),
        pltpu.VMEM_SHARED((MERGE,), jnp.float32),
        pltpu.VMEM_SHARED((MERGE,), jnp.int32),
        pltpu.SemaphoreType.DMA,
    ],
)(_sc_body)


@jax.jit
def kernel(x, W):
    res = _matvec(x, W)
    return _sc_topk_scatter(res.reshape(PAD_ROWS))


# speedup vs baseline: 1.0131x; 1.0054x over previous
import functools

import jax
import jax.numpy as jnp
from jax import lax
from jax.experimental import pallas as pl
from jax.experimental.pallas import tpu as pltpu
import jax.experimental.pallas.tpu_sc as plsc

NUM_ROWS = 100000
DIM = 4096
TOPK = 32

BLOCK_ROWS = 1024
NUM_BLOCKS = 98
PAD_ROWS = NUM_BLOCKS * BLOCK_ROWS
SUBS = PAD_ROWS // 128

NEG_INF = float("-inf")
BIG = 2 ** 30

NTILES = 16
CHUNK = PAD_ROWS // NTILES
CHUNK_PAD = 6400
NSLICE = CHUNK_PAD // 16
SMAX_PAD = 512
NGROUP = SMAX_PAD // 16
CAND = 48
MERGE = NTILES * CAND
MSLICE = MERGE // 16
MSMAX_PAD = 256

LAST_CHUNK = NUM_ROWS - (NTILES - 1) * CHUNK


def _matvec_kernel(x_ref, w_ref, out_ref):
    i = pl.program_id(0)
    y = lax.dot_general(
        w_ref[...].astype(jnp.bfloat16), x_ref[...].astype(jnp.bfloat16),
        dimension_numbers=(((1,), (0,)), ((), ())),
        preferred_element_type=jnp.float32,
    )
    y2 = y.reshape(BLOCK_ROWS // 128, 128)
    sub = lax.broadcasted_iota(jnp.int32, (BLOCK_ROWS // 128, 128), 0)
    lane = lax.broadcasted_iota(jnp.int32, (BLOCK_ROWS // 128, 128), 1)
    g = i * BLOCK_ROWS + sub * 128 + lane
    out_ref[...] = jnp.where(g < NUM_ROWS, y2, NEG_INF)


def _matvec(x, W):
    return pl.pallas_call(
        _matvec_kernel,
        grid=(NUM_BLOCKS,),
        in_specs=[
            pl.BlockSpec((DIM,), lambda i: (0,)),
            pl.BlockSpec((BLOCK_ROWS, DIM), lambda i: (i, 0)),
        ],
        out_specs=pl.BlockSpec((BLOCK_ROWS // 128, 128), lambda i: (i, 0)),
        out_shape=jax.ShapeDtypeStruct((SUBS, 128), jnp.float32),
    )(x, W)


def _lanes():
    return lax.iota(jnp.int32, 16)


def _splat_i32(v):
    return jnp.broadcast_to(v.astype(jnp.int32), (16,))


def _splat_f32(v):
    return jnp.broadcast_to(v.astype(jnp.float32), (16,))


def _neg16():
    return jnp.full((16,), NEG_INF, jnp.float32)


def _fill(ref, start, n_vregs, value):
    for j in range(n_vregs):
        ref[pl.ds(start + j * 16, 16)] = value


def _build_slice_max(src, dst, n_groups, unroll=1):
    lanes = _lanes()

    def body(j, _):
        base_ids = (j * 16 + lanes) * 16
        m = plsc.load_gather(src, [base_ids])
        for k in range(1, 16):
            m = jnp.maximum(m, plsc.load_gather(src, [base_ids + k]))
        dst[pl.ds(j * 16, 16)] = m
        return 0

    lax.fori_loop(0, n_groups, body, 0, unroll=unroll)


def _select_topk(buf, smax, l2, l2_vregs, n_iter, out_vals, out_inds, ind_of):
    lanes = _lanes()
    lane0 = lanes == 0

    def round_body(t, _):
        vs = [l2[pl.ds(j * 16, 16)] for j in range(l2_vregs)]
        m16 = vs[0]
        for v in vs[1:]:
            m16 = jnp.maximum(m16, v)
        m = jnp.max(m16)
        g_star = plsc.all_reduce_ffs(vs[0] == m)[0]
        for j in range(1, l2_vregs):
            fj = plsc.all_reduce_ffs(vs[j] == m)[0]
            g_star = jnp.where(g_star < 16, g_star, j * 16 + fj)

        vg = smax[pl.ds(g_star * 16, 16)]
        s_loc = plsc.all_reduce_ffs(vg == m)[0]
        s_star = g_star * 16 + s_loc

        vb = buf[pl.ds(s_star * 16, 16)]
        l_loc = plsc.all_reduce_ffs(vb == m)[0]
        flat = s_star * 16 + l_loc

        plsc.store_scatter(out_vals, [_splat_i32(t)], _splat_f32(m),
                           mask=lane0)
        plsc.store_scatter(out_inds, [_splat_i32(t)], ind_of(flat),
                           mask=lane0)

        plsc.store_scatter(buf, [_splat_i32(flat)], _neg16(), mask=lane0)
        new_smax = jnp.max(jnp.where(lanes == l_loc, NEG_INF, vb))
        plsc.store_scatter(smax, [_splat_i32(s_star)],
                           _splat_f32(new_smax), mask=lane0)
        new_l2 = jnp.max(jnp.where(lanes == s_loc, new_smax, vg))
        plsc.store_scatter(l2, [_splat_i32(g_star)],
                           _splat_f32(new_l2), mask=lane0)
        return 0

    lax.fori_loop(0, n_iter, round_body, 0, unroll=False)


def _sc_body(res_hbm, out_hbm, buf, zbuf, smax, l2, lvals, linds,
             mvals, minds, msmax, ml2, shv, shi, sem):
    c = lax.axis_index("c")
    w = lax.axis_index("s")
    base = w * CHUNK

    @pl.when(c == 0)
    def _local():
        cp = pltpu.async_copy(res_hbm.at[pl.ds(base, CHUNK)],
                              buf.at[pl.ds(0, CHUNK)], sem)
        zero16 = jnp.zeros((16,), jnp.float32)

        def zero_body(j, _):
            zbuf[pl.ds(j * 16, 16)] = zero16
            return 0

        lax.fori_loop(0, CHUNK_PAD // 16, zero_body, 0, unroll=8)
        _fill(buf, CHUNK, (CHUNK_PAD - CHUNK) // 16, _neg16())
        _fill(smax, 0, NGROUP, _neg16())
        _fill(l2, 0, NGROUP // 16, _neg16())
        _fill(lvals, 0, CAND // 16, _neg16())
        _fill(linds, 0, CAND // 16, jnp.full((16,), BIG, jnp.int32))
        cp.wait()

        _build_slice_max(buf, smax, NSLICE // 16, unroll=4)
        _build_slice_max(smax, l2, NGROUP // 16)
        _select_topk(buf, smax, l2, NGROUP // 16, TOPK + 1, lvals, linds,
                     lambda flat: _splat_i32(base + flat))

        pltpu.sync_copy(lvals, shv.at[pl.ds(w * CAND, CAND)])
        pltpu.sync_copy(linds, shi.at[pl.ds(w * CAND, CAND)])

    plsc.subcore_barrier()

    @pl.when(c == 0)
    def _merge_and_scatter():
        pltpu.sync_copy(shv, mvals)
        pltpu.sync_copy(shi, minds)
        _fill(msmax, 0, MSMAX_PAD // 16, _neg16())

        _build_slice_max(mvals, msmax, MSLICE // 16)
        _build_slice_max(msmax, ml2, 1)
        _select_topk(mvals, msmax, ml2, 1, TOPK + 1,
                     lvals, linds,
                     lambda flat: plsc.load_gather(minds, [_splat_i32(flat)]))

        v0 = lvals[pl.ds(0, 16)][0]
        vlast = lvals[pl.ds(TOPK, 16)][0]
        vlast16 = _splat_f32(vlast)
        inv16 = jnp.full((16,), 1.0, jnp.float32) / _splat_f32(v0 - vlast)

        for b in range(2):
            idx = linds[pl.ds(b * 16, 16)]
            val = (lvals[pl.ds(b * 16, 16)] - vlast16) * inv16
            local = idx - base
            ok = (local >= 0) & (local < CHUNK)
            safe = jnp.minimum(jnp.maximum(local, 0), CHUNK - 1)
            plsc.store_scatter(zbuf, [safe], val, mask=ok)

        @pl.when(w < NTILES - 1)
        def _():
            pltpu.sync_copy(zbuf.at[pl.ds(0, CHUNK)],
                            out_hbm.at[pl.ds(base, CHUNK)])

        @pl.when(w == NTILES - 1)
        def _():
            pltpu.sync_copy(zbuf.at[pl.ds(0, LAST_CHUNK)],
                            out_hbm.at[pl.ds(base, LAST_CHUNK)])


_sc_topk_scatter = functools.partial(
    pl.kernel,
    out_type=jax.ShapeDtypeStruct((NUM_ROWS,), jnp.float32),
    mesh=plsc.VectorSubcoreMesh(core_axis_name="c", subcore_axis_name="s",
                                num_cores=2, num_subcores=NTILES),
    compiler_params=pltpu.CompilerParams(needs_layout_passes=False),
    scratch_types=[
        pltpu.VMEM((CHUNK_PAD,), jnp.float32),
        pltpu.VMEM((CHUNK_PAD,), jnp.float32),
        pltpu.VMEM((SMAX_PAD,), jnp.float32),
        pltpu.VMEM((NGROUP,), jnp.float32),
        pltpu.VMEM((CAND,), jnp.float32),
        pltpu.VMEM((CAND,), jnp.int32),
        pltpu.VMEM((MERGE,), jnp.float32),
        pltpu.VMEM((MERGE,), jnp.int32),
        pltpu.VMEM((MSMAX_PAD,), jnp.float32),
        pltpu.VMEM((16,), jnp.float32),
        pltpu.VMEM_SHARED((MERGE,), jnp.float32),
        pltpu.VMEM_SHARED((MERGE,), jnp.int32),
        pltpu.SemaphoreType.DMA,
    ],
)(_sc_body)


@jax.jit
def kernel(x, W):
    res = _matvec(x, W)
    return _sc_topk_scatter(res.reshape(PAD_ROWS))


# speedup vs baseline: 1.0146x; 1.0015x over previous
import functools

import jax
import jax.numpy as jnp
from jax import lax
from jax.experimental import pallas as pl
from jax.experimental.pallas import tpu as pltpu
import jax.experimental.pallas.tpu_sc as plsc

NUM_ROWS = 100000
DIM = 4096
TOPK = 32

BLOCK_ROWS = 1024
NUM_BLOCKS = 98
PAD_ROWS = NUM_BLOCKS * BLOCK_ROWS
SUBS = PAD_ROWS // 128

NEG_INF = float("-inf")
BIG = 2 ** 30

NTILES = 16
CHUNK = PAD_ROWS // NTILES
CHUNK_PAD = 6400
NSLICE = CHUNK_PAD // 16
SMAX_PAD = 512
NGROUP = SMAX_PAD // 16
CAND = 48
MERGE = NTILES * CAND
MSLICE = MERGE // 16
MSMAX_PAD = 256

LAST_CHUNK = NUM_ROWS - (NTILES - 1) * CHUNK


def _matvec_kernel(x_ref, w_ref, out_ref):
    i = pl.program_id(0)
    y = lax.dot_general(
        w_ref[...].astype(jnp.bfloat16), x_ref[...].astype(jnp.bfloat16),
        dimension_numbers=(((1,), (0,)), ((), ())),
        preferred_element_type=jnp.float32,
    )
    y2 = y.reshape(BLOCK_ROWS // 128, 128)
    sub = lax.broadcasted_iota(jnp.int32, (BLOCK_ROWS // 128, 128), 0)
    lane = lax.broadcasted_iota(jnp.int32, (BLOCK_ROWS // 128, 128), 1)
    g = i * BLOCK_ROWS + sub * 128 + lane
    out_ref[...] = jnp.where(g < NUM_ROWS, y2, NEG_INF)


def _matvec(x, W):
    return pl.pallas_call(
        _matvec_kernel,
        grid=(NUM_BLOCKS,),
        in_specs=[
            pl.BlockSpec((DIM,), lambda i: (0,)),
            pl.BlockSpec((BLOCK_ROWS, DIM), lambda i: (i, 0)),
        ],
        out_specs=pl.BlockSpec((BLOCK_ROWS // 128, 128), lambda i: (i, 0)),
        out_shape=jax.ShapeDtypeStruct((SUBS, 128), jnp.float32),
    )(x, W)


def _lanes():
    return lax.iota(jnp.int32, 16)


def _splat_i32(v):
    return jnp.broadcast_to(v.astype(jnp.int32), (16,))


def _splat_f32(v):
    return jnp.broadcast_to(v.astype(jnp.float32), (16,))


def _neg16():
    return jnp.full((16,), NEG_INF, jnp.float32)


def _fill(ref, start, n_vregs, value):
    for j in range(n_vregs):
        ref[pl.ds(start + j * 16, 16)] = value


def _build_slice_max(src, dst, n_groups, unroll=1):
    lanes = _lanes()

    def body(j, _):
        base_ids = (j * 16 + lanes) * 16
        m = plsc.load_gather(src, [base_ids])
        for k in range(1, 16):
            m = jnp.maximum(m, plsc.load_gather(src, [base_ids + k]))
        dst[pl.ds(j * 16, 16)] = m
        return 0

    lax.fori_loop(0, n_groups, body, 0, unroll=unroll)


def _select_topk(buf, smax, l2, l2_vregs, n_iter, out_vals, out_inds, ind_of):
    lanes = _lanes()
    lane0 = lanes == 0

    def round_body(t, _):
        vs = [l2[pl.ds(j * 16, 16)] for j in range(l2_vregs)]
        m16 = vs[0]
        for v in vs[1:]:
            m16 = jnp.maximum(m16, v)
        m = jnp.max(m16)
        g_star = plsc.all_reduce_ffs(vs[0] == m)[0]
        for j in range(1, l2_vregs):
            fj = plsc.all_reduce_ffs(vs[j] == m)[0]
            g_star = jnp.where(g_star < 16, g_star, j * 16 + fj)

        vg = smax[pl.ds(g_star * 16, 16)]
        s_loc = plsc.all_reduce_ffs(vg == m)[0]
        s_star = g_star * 16 + s_loc

        vb = buf[pl.ds(s_star * 16, 16)]
        l_loc = plsc.all_reduce_ffs(vb == m)[0]
        flat = s_star * 16 + l_loc

        plsc.store_scatter(out_vals, [_splat_i32(t)], _splat_f32(m),
                           mask=lane0)
        plsc.store_scatter(out_inds, [_splat_i32(t)], ind_of(flat),
                           mask=lane0)

        plsc.store_scatter(buf, [_splat_i32(flat)], _neg16(), mask=lane0)
        new_smax = jnp.max(jnp.where(lanes == l_loc, NEG_INF, vb))
        plsc.store_scatter(smax, [_splat_i32(s_star)],
                           _splat_f32(new_smax), mask=lane0)
        new_l2 = jnp.max(jnp.where(lanes == s_loc, new_smax, vg))
        plsc.store_scatter(l2, [_splat_i32(g_star)],
                           _splat_f32(new_l2), mask=lane0)
        return 0

    lax.fori_loop(0, n_iter, round_body, 0, unroll=False)


def _sc_body(res_hbm, out_hbm, buf, zbuf, smax, l2, lvals, linds,
             mvals, minds, msmax, ml2, shv, shi, sem):
    c = lax.axis_index("c")
    w = lax.axis_index("s")
    base = w * CHUNK

    @pl.when(c == 0)
    def _local():
        cp = pltpu.async_copy(res_hbm.at[pl.ds(base, CHUNK)],
                              buf.at[pl.ds(0, CHUNK)], sem)
        zero16 = jnp.zeros((16,), jnp.float32)

        def zero_body(j, _):
            zbuf[pl.ds(j * 16, 16)] = zero16
            return 0

        lax.fori_loop(0, CHUNK_PAD // 16, zero_body, 0, unroll=8)
        _fill(buf, CHUNK, (CHUNK_PAD - CHUNK) // 16, _neg16())
        _fill(smax, 0, NGROUP, _neg16())
        _fill(l2, 0, NGROUP // 16, _neg16())
        _fill(lvals, 0, CAND // 16, _neg16())
        _fill(linds, 0, CAND // 16, jnp.full((16,), BIG, jnp.int32))
        cp.wait()

        _build_slice_max(buf, smax, NSLICE // 16, unroll=4)
        _build_slice_max(smax, l2, NGROUP // 16)
        _select_topk(buf, smax, l2, NGROUP // 16, TOPK + 1, lvals, linds,
                     lambda flat: _splat_i32(base + flat))

        d1 = pltpu.async_copy(lvals, shv.at[pl.ds(w * CAND, CAND)], sem)
        d2 = pltpu.async_copy(linds, shi.at[pl.ds(w * CAND, CAND)], sem)
        d1.wait()
        d2.wait()

    plsc.subcore_barrier()

    @pl.when(c == 0)
    def _merge_and_scatter():
        d1 = pltpu.async_copy(shv, mvals, sem)
        d2 = pltpu.async_copy(shi, minds, sem)
        _fill(msmax, 0, MSMAX_PAD // 16, _neg16())
        d1.wait()
        d2.wait()

        _build_slice_max(mvals, msmax, MSLICE // 16)
        _build_slice_max(msmax, ml2, 1)
        _select_topk(mvals, msmax, ml2, 1, TOPK + 1,
                     lvals, linds,
                     lambda flat: plsc.load_gather(minds, [_splat_i32(flat)]))

        v0 = lvals[pl.ds(0, 16)][0]
        vlast = lvals[pl.ds(TOPK, 16)][0]
        vlast16 = _splat_f32(vlast)
        inv16 = jnp.full((16,), 1.0, jnp.float32) / _splat_f32(v0 - vlast)

        for b in range(2):
            idx = linds[pl.ds(b * 16, 16)]
            val = (lvals[pl.ds(b * 16, 16)] - vlast16) * inv16
            local = idx - base
            ok = (local >= 0) & (local < CHUNK)
            safe = jnp.minimum(jnp.maximum(local, 0), CHUNK - 1)
            plsc.store_scatter(zbuf, [safe], val, mask=ok)

        @pl.when(w < NTILES - 1)
        def _():
            pltpu.sync_copy(zbuf.at[pl.ds(0, CHUNK)],
                            out_hbm.at[pl.ds(base, CHUNK)])

        @pl.when(w == NTILES - 1)
        def _():
            pltpu.sync_copy(zbuf.at[pl.ds(0, LAST_CHUNK)],
                            out_hbm.at[pl.ds(base, LAST_CHUNK)])


_sc_topk_scatter = functools.partial(
    pl.kernel,
    out_type=jax.ShapeDtypeStruct((NUM_ROWS,), jnp.float32),
    mesh=plsc.VectorSubcoreMesh(core_axis_name="c", subcore_axis_name="s",
                                num_cores=2, num_subcores=NTILES),
    compiler_params=pltpu.CompilerParams(needs_layout_passes=False),
    scratch_types=[
        pltpu.VMEM((CHUNK_PAD,), jnp.float32),
        pltpu.VMEM((CHUNK_PAD,), jnp.float32),
        pltpu.VMEM((SMAX_PAD,), jnp.float32),
        pltpu.VMEM((NGROUP,), jnp.float32),
        pltpu.VMEM((CAND,), jnp.float32),
        pltpu.VMEM((CAND,), jnp.int32),
        pltpu.VMEM((MERGE,), jnp.float32),
        pltpu.VMEM((MERGE,), jnp.int32),
        pltpu.VMEM((MSMAX_PAD,), jnp.float32),
        pltpu.VMEM((16,), jnp.float32),
        pltpu.VMEM_SHARED((MERGE,), jnp.float32),
        pltpu.VMEM_SHARED((MERGE,), jnp.int32),
        pltpu.SemaphoreType.DMA,
    ],
)(_sc_body)


@jax.jit
def kernel(x, W):
    res = _matvec(x, W)
    return _sc_topk_scatter(res.reshape(PAD_ROWS))
